# baseline (speedup 1.0000x reference)
import functools

import jax
import jax.numpy as jnp
from jax import lax
from jax.experimental import pallas as pl
from jax.experimental.pallas import tpu as pltpu
from jax.experimental.pallas import tpu_sc as plsc

_N = 50000
_E = 800000

_IW = 256
_MW = 32
_ROWS = 3200
_EPAD = _ROWS * _IW
_NTILES = 16
_SB = 10
_NCHUNK = _SB
_TROWS_A = _ROWS // _NTILES
_BW_SPLIT = 640
_TROWS_B0 = _BW_SPLIT // _NTILES
_TROWS_B1 = (_ROWS - _BW_SPLIT) // _NTILES
_ACC_ROWS = 50176
_TSPAN = _ACC_ROWS // _NTILES
_TRASH = _N

_RBLK = 1000


def _mlp_body(fw_ref, bw_ref, bd_ref, w1_ref, b1_ref, w2_ref, b2_ref,
              w3_ref, b3_ref, mbd_ref, mfw_ref, mbw_ref, selfo_ref):
    x = jnp.concatenate([fw_ref[...], bw_ref[...], bd_ref[...]], axis=1)
    h = jnp.dot(x, w1_ref[...], preferred_element_type=jnp.float32) + b1_ref[...]
    h = jnp.where(h > 0, h, 0.2 * h)
    h = jnp.dot(h, w2_ref[...], preferred_element_type=jnp.float32) + b2_ref[...]
    h = jnp.where(h > 0, h, 0.2 * h)
    h = jnp.dot(h, w3_ref[...], preferred_element_type=jnp.float32) + b3_ref[...]
    kf = jax.nn.sigmoid(h[:, 0:1])
    kb = jax.nn.sigmoid(h[:, 1:2])
    mbd_ref[...] = h[:, 32:64] * kf + h[:, 96:128] * kb
    mfw_ref[...] = h[:, 64:96] * kf
    mbw_ref[...] = h[:, 128:160] * kb
    selfo_ref[...] = h[:, 160:256]


def _node_mlps(fw_nf, bw_nf, bd_nf, W1, B1, W2, B2, W3, B3):
    f32 = jnp.float32

    def row(w):
        return pl.BlockSpec((_RBLK, w), lambda i: (i, 0))

    def full(a, b):
        return pl.BlockSpec((a, b), lambda i: (0, 0))

    return pl.pallas_call(
        _mlp_body,
        grid=(_N // _RBLK,),
        in_specs=[row(64), row(32), row(32),
                  full(128, 192), full(1, 192),
                  full(192, 192), full(1, 192),
                  full(192, 256), full(1, 256)],
        out_specs=[row(_MW), row(_MW), row(_MW), row(96)],
        out_shape=[jax.ShapeDtypeStruct((_N, _MW), f32),
                   jax.ShapeDtypeStruct((_N, _MW), f32),
                   jax.ShapeDtypeStruct((_N, _MW), f32),
                   jax.ShapeDtypeStruct((_N, 96), f32)],
    )(fw_nf, bw_nf, bd_nf, W1, B1, W2, B2, W3, B3)


def _combine_body(selfo_ref, acc_ref, rden_ref, obd_ref, obw_ref, ofw_ref):
    s = selfo_ref[...]
    a = acc_ref[...]
    r = rden_ref[...]
    obd_ref[...] = jax.nn.sigmoid(s[:, 0:32] + a[:, 0:32] * r)
    obw_ref[...] = jax.nn.sigmoid(
        s[:, 32:64] + (a[:, 64:96] + a[:, 96:128]) * r)
    ofw_ref[...] = jax.nn.sigmoid(s[:, 64:96] + a[:, 32:64] * r)


def _combine(selfo, accpack, rden):
    f32 = jnp.float32

    def row(w):
        return pl.BlockSpec((_RBLK, w), lambda i: (i, 0))

    return pl.pallas_call(
        _combine_body,
        grid=(_N // _RBLK,),
        in_specs=[row(96), row(128), row(32)],
        out_specs=[row(32), row(32), row(32)],
        out_shape=[jax.ShapeDtypeStruct((_N, 32), f32)] * 3,
    )(selfo, accpack, rden)


def _sc_body(mbd_hbm, mfw_hbm, mbw_hbm, src_hbm, dst_hbm, z2_hbm, z1_hbm,
             apack_hbm, deg0_hbm, deg1_hbm,
             acc_s, deg_s, sidx_v, didx_v, rows_v, ones_v, gsem, ssem):
    cid = lax.axis_index("c")
    sid = lax.axis_index("s")

    one16 = jnp.ones((16,), jnp.float32)

    def _init_ones(i, c):
        ones_v[pl.ds(i * 16, 16)] = one16
        return c

    lax.fori_loop(0, 8, _init_ones, 0)

    def _zero_acc():
        base = sid * _TSPAN
        pltpu.sync_copy(z2_hbm.at[pl.ds(base, _TSPAN)],
                        acc_s.at[pl.ds(base, _TSPAN)])

    def _zero_deg():
        base = sid * _TSPAN
        pltpu.sync_copy(z1_hbm.at[pl.ds(base, _TSPAN)],
                        deg_s.at[pl.ds(base, _TSPAN)])

    def _edge_pass(msg_hbm, pass_base, trows, deg_half):
        tile_base = pass_base + sid * trows
        nsup = trows // _SB

        def fire_g(c):
            b = lax.rem(c, 2)
            pltpu.async_copy(msg_hbm.at[sidx_v.at[c]], rows_v.at[b], gsem)

        def drain_g(c):
            b = lax.rem(c, 2)
            pltpu.make_async_copy(msg_hbm.at[sidx_v.at[c]],
                                  rows_v.at[b], gsem).wait()

        def fire_s(c, deg_on):
            b = lax.rem(c, 2)
            pltpu.async_copy(rows_v.at[b], acc_s.at[didx_v.at[c]],
                             ssem, add=True)

            @pl.when(deg_on)
            def _():
                pltpu.async_copy(ones_v, deg_s.at[didx_v.at[c, pl.ds(0, 128)]],
                                 ssem, add=True)
                pltpu.async_copy(ones_v,
                                 deg_s.at[didx_v.at[c, pl.ds(128, 128)]],
                                 ssem, add=True)

        def drain_s(c, deg_on):
            b = lax.rem(c, 2)
            pltpu.make_async_copy(rows_v.at[b], acc_s.at[didx_v.at[c]],
                                  ssem).wait()

            @pl.when(deg_on)
            def _():
                pltpu.make_async_copy(
                    ones_v, deg_s.at[didx_v.at[c, pl.ds(0, 128)]],
                    ssem).wait()
                pltpu.make_async_copy(
                    ones_v, deg_s.at[didx_v.at[c, pl.ds(128, 128)]],
                    ssem).wait()

        def superchunk(sc, carry):
            r0 = tile_base + sc * _SB
            if deg_half == 0:
                deg_on = jnp.bool_(False)
            elif deg_half == 1:
                deg_on = sc < nsup // 2
            else:
                deg_on = sc >= nsup // 2
            pltpu.sync_copy(src_hbm.at[pl.ds(r0, _SB)], sidx_v)
            pltpu.sync_copy(dst_hbm.at[pl.ds(r0, _SB)], didx_v)

            def inner(c, cc):
                @pl.when(c >= 2)
                def _():
                    drain_s(c - 2, deg_on)

                fire_g(c)

                @pl.when(c >= 1)
                def _():
                    drain_g(c - 1)
                    fire_s(c - 1, deg_on)

                return cc

            lax.fori_loop(0, _NCHUNK, inner, 0)
            drain_g(_NCHUNK - 1)
            fire_s(_NCHUNK - 1, deg_on)
            drain_s(_NCHUNK - 2, deg_on)
            drain_s(_NCHUNK - 1, deg_on)
            return carry

        lax.fori_loop(0, nsup, superchunk, 0)

    def _flush(grp):
        base = sid * _TSPAN
        pltpu.sync_copy(acc_s.at[pl.ds(base, _TSPAN)],
                        apack_hbm.at[pl.ds(base, _TSPAN), pl.ds(32 * grp, 32)])

    def _flush_deg(deg_hbm):
        base = sid * _TSPAN
        pltpu.sync_copy(deg_s.at[pl.ds(base, _TSPAN)],
                        deg_hbm.at[pl.ds(base, _TSPAN)])

    @pl.when(cid == 0)
    def _core0():
        _zero_acc()
        _zero_deg()
        plsc.subcore_barrier()
        _edge_pass(mbd_hbm, 0, _TROWS_A, 1)
        plsc.subcore_barrier()
        _flush(0)
        _flush_deg(deg0_hbm)
        plsc.subcore_barrier()
        _zero_acc()
        plsc.subcore_barrier()
        _edge_pass(mbw_hbm, 0, _TROWS_B0, 0)
        plsc.subcore_barrier()
        _flush(2)

    @pl.when(cid == 1)
    def _core1():
        _zero_acc()
        _zero_deg()
        plsc.subcore_barrier()
        _edge_pass(mfw_hbm, 0, _TROWS_A, 2)
        plsc.subcore_barrier()
        _flush(1)
        _flush_deg(deg1_hbm)
        plsc.subcore_barrier()
        _zero_acc()
        plsc.subcore_barrier()
        _edge_pass(mbw_hbm, _BW_SPLIT, _TROWS_B1, 0)
        plsc.subcore_barrier()
        _flush(3)


@functools.lru_cache(maxsize=1)
def _sc_scatter_call():
    mesh = plsc.VectorSubcoreMesh(core_axis_name="c", subcore_axis_name="s",
                                  num_cores=2, num_subcores=_NTILES)
    return pl.kernel(
        _sc_body,
        out_type=[jax.ShapeDtypeStruct((_ACC_ROWS, 128), jnp.float32),
                  jax.ShapeDtypeStruct((_ACC_ROWS,), jnp.float32),
                  jax.ShapeDtypeStruct((_ACC_ROWS,), jnp.float32)],
        mesh=mesh,
        scratch_types=[
            pltpu.VMEM_SHARED((_ACC_ROWS, _MW), jnp.float32),
            pltpu.VMEM_SHARED((_ACC_ROWS,), jnp.float32),
            pltpu.VMEM((_SB, _IW), jnp.int32),
            pltpu.VMEM((_SB, _IW), jnp.int32),
            pltpu.VMEM((2, _IW, _MW), jnp.float32),
            pltpu.VMEM((128,), jnp.float32),
            pltpu.SemaphoreType.DMA,
            pltpu.SemaphoreType.DMA,
        ],
        compiler_params=pltpu.CompilerParams(use_tc_tiling_on_sc=False,
                                            needs_layout_passes=False),
    )


def _assemble_weights(params):
    f32 = jnp.float32
    (wf1, bf1), (wf2, bf2), (wf3, bf3) = params["fw"]
    (wb1, bb1), (wb2, bb2), (wb3, bb3) = params["bw"]
    (ws1, bs1), (ws2, bs2), (ws3, bs3) = params["self"]

    W1 = jnp.zeros((128, 192), f32)
    W1 = W1.at[0:64, 0:64].set(wf1[0:64])
    W1 = W1.at[96:128, 0:64].set(wf1[64:96])
    W1 = W1.at[64:96, 64:128].set(wb1[0:32])
    W1 = W1.at[96:128, 64:128].set(wb1[32:64])
    W1 = W1.at[:, 128:192].set(ws1)
    B1 = jnp.concatenate([bf1, bb1, bs1]).reshape(1, 192)

    W2 = jnp.zeros((192, 192), f32)
    W2 = W2.at[0:64, 0:64].set(wf2)
    W2 = W2.at[64:128, 64:128].set(wb2)
    W2 = W2.at[128:192, 128:192].set(ws2)
    B2 = jnp.concatenate([bf2, bb2, bs2]).reshape(1, 192)

    W3 = jnp.zeros((192, 256), f32)
    W3 = W3.at[0:64, 0:1].set(wf3[:, 0:1])
    W3 = W3.at[0:64, 32:64].set(wf3[:, 1:33])
    W3 = W3.at[0:64, 64:96].set(wf3[:, 33:65])
    W3 = W3.at[64:128, 1:2].set(wb3[:, 0:1])
    W3 = W3.at[64:128, 96:128].set(wb3[:, 1:33])
    W3 = W3.at[64:128, 128:160].set(wb3[:, 33:65])
    W3 = W3.at[128:192, 160:256].set(ws3)
    B3 = jnp.zeros((256,), f32)
    B3 = B3.at[0:1].set(bf3[0:1])
    B3 = B3.at[32:64].set(bf3[1:33])
    B3 = B3.at[64:96].set(bf3[33:65])
    B3 = B3.at[1:2].set(bb3[0:1])
    B3 = B3.at[96:128].set(bb3[1:33])
    B3 = B3.at[128:160].set(bb3[33:65])
    B3 = B3.at[160:256].set(bs3)
    B3 = B3.reshape(1, 256)
    return W1, B1, W2, B2, W3, B3


def kernel(bd_nf, bw_nf, fw_nf, edge_index, params):
    W1, B1, W2, B2, W3, B3 = _assemble_weights(params)

    src = edge_index[0].astype(jnp.int32)
    dst = edge_index[1].astype(jnp.int32)
    pad = _EPAD - _E
    src3d = jnp.concatenate(
        [src, jnp.zeros((pad,), jnp.int32)]).reshape(_ROWS, _IW)
    dst3d = jnp.concatenate(
        [dst, jnp.full((pad,), _TRASH, jnp.int32)]).reshape(_ROWS, _IW)

    mbd, mfw, mbw, selfo = _node_mlps(fw_nf, bw_nf, bd_nf,
                                      W1, B1, W2, B2, W3, B3)
    zeros_acc = jnp.zeros((_ACC_ROWS, _MW), jnp.float32)
    zeros_deg = jnp.zeros((_ACC_ROWS,), jnp.float32)
    accpack, deg0, deg1 = _sc_scatter_call()(mbd, mfw, mbw, src3d, dst3d,
                                             zeros_acc, zeros_deg)
    rden = jnp.broadcast_to(
        (1.0 / jnp.maximum(deg0 + deg1, 1.0))[:, None], (_ACC_ROWS, 32))
    out_bd, out_bw, out_fw = _combine(selfo, accpack, rden)
    return (out_bd, out_bw, out_fw)


# speedup vs baseline: 1.5760x; 1.5760x over previous
import functools

import jax
import jax.numpy as jnp
from jax import lax
from jax.experimental import pallas as pl
from jax.experimental.pallas import tpu as pltpu
from jax.experimental.pallas import tpu_sc as plsc

_N = 50000
_E = 800000

_IW = 256
_MW = 32
_ROWS = 3200
_EPAD = _ROWS * _IW
_NTILES = 16
_SB = 10
_NCHUNK = _SB
_TROWS_A = _ROWS // _NTILES
_BW_SPLIT = 1600
_TROWS_B0 = _BW_SPLIT // _NTILES
_TROWS_B1 = (_ROWS - _BW_SPLIT) // _NTILES
_ACC_ROWS = 50176
_TSPAN = _ACC_ROWS // _NTILES
_TRASH = _N

_RBLK = 1000


def _mlp_body(fw_ref, bw_ref, bd_ref, w1_ref, b1_ref, w2_ref, b2_ref,
              w3_ref, b3_ref, mbd_ref, mfw_ref, mbw_ref, selfo_ref):
    x = jnp.concatenate([fw_ref[...], bw_ref[...], bd_ref[...]], axis=1)
    h = jnp.dot(x, w1_ref[...], preferred_element_type=jnp.float32) + b1_ref[...]
    h = jnp.where(h > 0, h, 0.2 * h)
    h = jnp.dot(h, w2_ref[...], preferred_element_type=jnp.float32) + b2_ref[...]
    h = jnp.where(h > 0, h, 0.2 * h)
    h = jnp.dot(h, w3_ref[...], preferred_element_type=jnp.float32) + b3_ref[...]
    kf = jax.nn.sigmoid(h[:, 0:1])
    kb = jax.nn.sigmoid(h[:, 1:2])
    mbd_ref[...] = h[:, 32:64] * kf + h[:, 96:128] * kb
    mfw_ref[...] = h[:, 64:96] * kf
    mbw_ref[...] = h[:, 128:160] * kb
    selfo_ref[...] = h[:, 160:256]


def _node_mlps(fw_nf, bw_nf, bd_nf, W1, B1, W2, B2, W3, B3):
    f32 = jnp.float32

    def row(w):
        return pl.BlockSpec((_RBLK, w), lambda i: (i, 0))

    def full(a, b):
        return pl.BlockSpec((a, b), lambda i: (0, 0))

    return pl.pallas_call(
        _mlp_body,
        grid=(_N // _RBLK,),
        in_specs=[row(64), row(32), row(32),
                  full(128, 192), full(1, 192),
                  full(192, 192), full(1, 192),
                  full(192, 256), full(1, 256)],
        out_specs=[row(_MW), row(_MW), row(_MW), row(96)],
        out_shape=[jax.ShapeDtypeStruct((_N, _MW), f32),
                   jax.ShapeDtypeStruct((_N, _MW), f32),
                   jax.ShapeDtypeStruct((_N, _MW), f32),
                   jax.ShapeDtypeStruct((_N, 96), f32)],
    )(fw_nf, bw_nf, bd_nf, W1, B1, W2, B2, W3, B3)


def _combine_body(selfo_ref, acc_ref, rden_ref, obd_ref, obw_ref, ofw_ref):
    s = selfo_ref[...]
    a = acc_ref[...]
    r = rden_ref[...]
    obd_ref[...] = jax.nn.sigmoid(s[:, 0:32] + a[:, 0:32] * r)
    obw_ref[...] = jax.nn.sigmoid(
        s[:, 32:64] + (a[:, 64:96] + a[:, 96:128]) * r)
    ofw_ref[...] = jax.nn.sigmoid(s[:, 64:96] + a[:, 32:64] * r)


def _combine(selfo, accpack, rden):
    f32 = jnp.float32

    def row(w):
        return pl.BlockSpec((_RBLK, w), lambda i: (i, 0))

    return pl.pallas_call(
        _combine_body,
        grid=(_N // _RBLK,),
        in_specs=[row(96), row(128), row(32)],
        out_specs=[row(32), row(32), row(32)],
        out_shape=[jax.ShapeDtypeStruct((_N, 32), f32)] * 3,
    )(selfo, accpack, rden)


def _sc_body(mbd_hbm, mfw_hbm, mbw_hbm, src_hbm, dst_hbm, z2_hbm, z1_hbm,
             apack_hbm, deg0_hbm, deg1_hbm,
             acc_s, deg_s, sidx_v, didx_v, rows_v, ones_v, gsem, ssem):
    cid = lax.axis_index("c")
    sid = lax.axis_index("s")

    one16 = jnp.ones((16,), jnp.float32)

    def _init_ones(i, c):
        ones_v[pl.ds(i * 16, 16)] = one16
        return c

    lax.fori_loop(0, 8, _init_ones, 0)

    def _zero_acc():
        base = sid * _TSPAN
        pltpu.sync_copy(z2_hbm.at[pl.ds(base, _TSPAN)],
                        acc_s.at[pl.ds(base, _TSPAN)])

    def _zero_deg():
        base = sid * _TSPAN
        pltpu.sync_copy(z1_hbm.at[pl.ds(base, _TSPAN)],
                        deg_s.at[pl.ds(base, _TSPAN)])

    def _edge_pass(msg_hbm, pass_base, trows, deg_half):
        tile_base = pass_base + sid * trows
        nsup = trows // _SB

        def fire_g(c):
            b = lax.rem(c, 2)
            pltpu.async_copy(msg_hbm.at[sidx_v.at[c]], rows_v.at[b], gsem)

        def drain_g(c):
            b = lax.rem(c, 2)
            pltpu.make_async_copy(msg_hbm.at[sidx_v.at[c]],
                                  rows_v.at[b], gsem).wait()

        def fire_s(c, deg_on):
            b = lax.rem(c, 2)
            pltpu.async_copy(rows_v.at[b], acc_s.at[didx_v.at[c]],
                             ssem, add=True)

            @pl.when(deg_on)
            def _():
                pltpu.async_copy(ones_v, deg_s.at[didx_v.at[c, pl.ds(0, 128)]],
                                 ssem, add=True)
                pltpu.async_copy(ones_v,
                                 deg_s.at[didx_v.at[c, pl.ds(128, 128)]],
                                 ssem, add=True)

        def drain_s(c, deg_on):
            b = lax.rem(c, 2)
            pltpu.make_async_copy(rows_v.at[b], acc_s.at[didx_v.at[c]],
                                  ssem).wait()

            @pl.when(deg_on)
            def _():
                pltpu.make_async_copy(
                    ones_v, deg_s.at[didx_v.at[c, pl.ds(0, 128)]],
                    ssem).wait()
                pltpu.make_async_copy(
                    ones_v, deg_s.at[didx_v.at[c, pl.ds(128, 128)]],
                    ssem).wait()

        def superchunk(sc, carry):
            r0 = tile_base + sc * _SB
            if deg_half == 0:
                deg_on = jnp.bool_(False)
            elif deg_half == 1:
                deg_on = sc < nsup // 2
            else:
                deg_on = sc >= nsup // 2
            pltpu.sync_copy(src_hbm.at[pl.ds(r0, _SB)], sidx_v)
            pltpu.sync_copy(dst_hbm.at[pl.ds(r0, _SB)], didx_v)

            def inner(c, cc):
                @pl.when(c >= 2)
                def _():
                    drain_s(c - 2, deg_on)

                fire_g(c)

                @pl.when(c >= 1)
                def _():
                    drain_g(c - 1)
                    fire_s(c - 1, deg_on)

                return cc

            lax.fori_loop(0, _NCHUNK, inner, 0)
            drain_g(_NCHUNK - 1)
            fire_s(_NCHUNK - 1, deg_on)
            drain_s(_NCHUNK - 2, deg_on)
            drain_s(_NCHUNK - 1, deg_on)
            return carry

        lax.fori_loop(0, nsup, superchunk, 0)

    def _flush(grp):
        base = sid * _TSPAN
        pltpu.sync_copy(acc_s.at[pl.ds(base, _TSPAN)],
                        apack_hbm.at[pl.ds(base, _TSPAN), pl.ds(32 * grp, 32)])

    def _flush_deg(deg_hbm):
        base = sid * _TSPAN
        pltpu.sync_copy(deg_s.at[pl.ds(base, _TSPAN)],
                        deg_hbm.at[pl.ds(base, _TSPAN)])

    @pl.when(cid == 0)
    def _core0():
        _zero_acc()
        _zero_deg()
        plsc.subcore_barrier()
        _edge_pass(mbd_hbm, 0, _TROWS_A, 1)
        plsc.subcore_barrier()
        _flush(0)
        _flush_deg(deg0_hbm)
        plsc.subcore_barrier()
        _zero_acc()
        plsc.subcore_barrier()
        _edge_pass(mbw_hbm, 0, _TROWS_B0, 0)
        plsc.subcore_barrier()
        _flush(2)

    @pl.when(cid == 1)
    def _core1():
        _zero_acc()
        _zero_deg()
        plsc.subcore_barrier()
        _edge_pass(mfw_hbm, 0, _TROWS_A, 2)
        plsc.subcore_barrier()
        _flush(1)
        _flush_deg(deg1_hbm)
        plsc.subcore_barrier()
        _zero_acc()
        plsc.subcore_barrier()
        _edge_pass(mbw_hbm, _BW_SPLIT, _TROWS_B1, 0)
        plsc.subcore_barrier()
        _flush(3)


@functools.lru_cache(maxsize=1)
def _sc_scatter_call():
    mesh = plsc.VectorSubcoreMesh(core_axis_name="c", subcore_axis_name="s",
                                  num_cores=2, num_subcores=_NTILES)
    return pl.kernel(
        _sc_body,
        out_type=[jax.ShapeDtypeStruct((_ACC_ROWS, 128), jnp.float32),
                  jax.ShapeDtypeStruct((_ACC_ROWS,), jnp.float32),
                  jax.ShapeDtypeStruct((_ACC_ROWS,), jnp.float32)],
        mesh=mesh,
        scratch_types=[
            pltpu.VMEM_SHARED((_ACC_ROWS, _MW), jnp.float32),
            pltpu.VMEM_SHARED((_ACC_ROWS,), jnp.float32),
            pltpu.VMEM((_SB, _IW), jnp.int32),
            pltpu.VMEM((_SB, _IW), jnp.int32),
            pltpu.VMEM((2, _IW, _MW), jnp.float32),
            pltpu.VMEM((128,), jnp.float32),
            pltpu.SemaphoreType.DMA,
            pltpu.SemaphoreType.DMA,
        ],
        compiler_params=pltpu.CompilerParams(use_tc_tiling_on_sc=False,
                                            needs_layout_passes=False),
    )


def _assemble_weights(params):
    f32 = jnp.float32
    (wf1, bf1), (wf2, bf2), (wf3, bf3) = params["fw"]
    (wb1, bb1), (wb2, bb2), (wb3, bb3) = params["bw"]
    (ws1, bs1), (ws2, bs2), (ws3, bs3) = params["self"]

    W1 = jnp.zeros((128, 192), f32)
    W1 = W1.at[0:64, 0:64].set(wf1[0:64])
    W1 = W1.at[96:128, 0:64].set(wf1[64:96])
    W1 = W1.at[64:96, 64:128].set(wb1[0:32])
    W1 = W1.at[96:128, 64:128].set(wb1[32:64])
    W1 = W1.at[:, 128:192].set(ws1)
    B1 = jnp.concatenate([bf1, bb1, bs1]).reshape(1, 192)

    W2 = jnp.zeros((192, 192), f32)
    W2 = W2.at[0:64, 0:64].set(wf2)
    W2 = W2.at[64:128, 64:128].set(wb2)
    W2 = W2.at[128:192, 128:192].set(ws2)
    B2 = jnp.concatenate([bf2, bb2, bs2]).reshape(1, 192)

    W3 = jnp.zeros((192, 256), f32)
    W3 = W3.at[0:64, 0:1].set(wf3[:, 0:1])
    W3 = W3.at[0:64, 32:64].set(wf3[:, 1:33])
    W3 = W3.at[0:64, 64:96].set(wf3[:, 33:65])
    W3 = W3.at[64:128, 1:2].set(wb3[:, 0:1])
    W3 = W3.at[64:128, 96:128].set(wb3[:, 1:33])
    W3 = W3.at[64:128, 128:160].set(wb3[:, 33:65])
    W3 = W3.at[128:192, 160:256].set(ws3)
    B3 = jnp.zeros((256,), f32)
    B3 = B3.at[0:1].set(bf3[0:1])
    B3 = B3.at[32:64].set(bf3[1:33])
    B3 = B3.at[64:96].set(bf3[33:65])
    B3 = B3.at[1:2].set(bb3[0:1])
    B3 = B3.at[96:128].set(bb3[1:33])
    B3 = B3.at[128:160].set(bb3[33:65])
    B3 = B3.at[160:256].set(bs3)
    B3 = B3.reshape(1, 256)
    return W1, B1, W2, B2, W3, B3


def kernel(bd_nf, bw_nf, fw_nf, edge_index, params):
    W1, B1, W2, B2, W3, B3 = _assemble_weights(params)

    src = edge_index[0].astype(jnp.int32)
    dst = edge_index[1].astype(jnp.int32)
    pad = _EPAD - _E
    pad_i = jnp.arange(pad, dtype=jnp.int32)
    src3d = jnp.concatenate(
        [src, pad_i % _N]).reshape(_ROWS, _IW)
    dst3d = jnp.concatenate(
        [dst, _TRASH + pad_i % (_ACC_ROWS - _N)]).reshape(_ROWS, _IW)

    mbd, mfw, mbw, selfo = _node_mlps(fw_nf, bw_nf, bd_nf,
                                      W1, B1, W2, B2, W3, B3)
    zeros_acc = jnp.zeros((_ACC_ROWS, _MW), jnp.float32)
    zeros_deg = jnp.zeros((_ACC_ROWS,), jnp.float32)
    accpack, deg0, deg1 = _sc_scatter_call()(mbd, mfw, mbw, src3d, dst3d,
                                             zeros_acc, zeros_deg)
    rden = jnp.broadcast_to(
        (1.0 / jnp.maximum(deg0 + deg1, 1.0))[:, None], (_ACC_ROWS, 32))
    out_bd, out_bw, out_fw = _combine(selfo, accpack, rden)
    return (out_bd, out_bw, out_fw)


# speedup vs baseline: 1.6379x; 1.0393x over previous
import functools

import jax
import jax.numpy as jnp
from jax import lax
from jax.experimental import pallas as pl
from jax.experimental.pallas import tpu as pltpu
from jax.experimental.pallas import tpu_sc as plsc

_N = 50000
_E = 800000

_IW = 256
_MW = 32
_ROWS = 3200
_EPAD = _ROWS * _IW
_NTILES = 16
_SB = 10
_NCHUNK = _SB
_TROWS_A = _ROWS // _NTILES
_BW_SPLIT = 1600
_TROWS_B0 = _BW_SPLIT // _NTILES
_TROWS_B1 = (_ROWS - _BW_SPLIT) // _NTILES
_ACC_ROWS = 50176
_TSPAN = _ACC_ROWS // _NTILES
_TRASH = _N

_RBLK = 2000


def _mlp_body(fw_ref, bw_ref, bd_ref, w1_ref, b1_ref, w2_ref, b2_ref,
              w3_ref, b3_ref, mbd_ref, mfw_ref, mbw_ref, selfo_ref):
    x = jnp.concatenate([fw_ref[...], bw_ref[...], bd_ref[...]], axis=1)
    h = jnp.dot(x, w1_ref[...], preferred_element_type=jnp.float32) + b1_ref[...]
    h = jnp.where(h > 0, h, 0.2 * h)
    h = jnp.dot(h, w2_ref[...], preferred_element_type=jnp.float32) + b2_ref[...]
    h = jnp.where(h > 0, h, 0.2 * h)
    h = jnp.dot(h, w3_ref[...], preferred_element_type=jnp.float32) + b3_ref[...]
    kf = jax.nn.sigmoid(h[:, 0:1])
    kb = jax.nn.sigmoid(h[:, 1:2])
    mbd_ref[...] = h[:, 32:64] * kf + h[:, 96:128] * kb
    mfw_ref[...] = h[:, 64:96] * kf
    mbw_ref[...] = h[:, 128:160] * kb
    selfo_ref[...] = h[:, 160:256]


def _node_mlps(fw_nf, bw_nf, bd_nf, W1, B1, W2, B2, W3, B3):
    f32 = jnp.float32

    def row(w):
        return pl.BlockSpec((_RBLK, w), lambda i: (i, 0))

    def full(a, b):
        return pl.BlockSpec((a, b), lambda i: (0, 0))

    return pl.pallas_call(
        _mlp_body,
        grid=(_N // _RBLK,),
        in_specs=[row(64), row(32), row(32),
                  full(128, 192), full(1, 192),
                  full(192, 192), full(1, 192),
                  full(192, 256), full(1, 256)],
        out_specs=[row(_MW), row(_MW), row(_MW), row(96)],
        out_shape=[jax.ShapeDtypeStruct((_N, _MW), f32),
                   jax.ShapeDtypeStruct((_N, _MW), f32),
                   jax.ShapeDtypeStruct((_N, _MW), f32),
                   jax.ShapeDtypeStruct((_N, 96), f32)],
    )(fw_nf, bw_nf, bd_nf, W1, B1, W2, B2, W3, B3)


def _combine_body(selfo_ref, acc_ref, rden_ref, obd_ref, obw_ref, ofw_ref):
    s = selfo_ref[...]
    a = acc_ref[...]
    r = rden_ref[...]
    obd_ref[...] = jax.nn.sigmoid(s[:, 0:32] + a[:, 0:32] * r)
    obw_ref[...] = jax.nn.sigmoid(
        s[:, 32:64] + (a[:, 64:96] + a[:, 96:128]) * r)
    ofw_ref[...] = jax.nn.sigmoid(s[:, 64:96] + a[:, 32:64] * r)


def _combine(selfo, accpack, rden):
    f32 = jnp.float32

    def row(w):
        return pl.BlockSpec((_RBLK, w), lambda i: (i, 0))

    return pl.pallas_call(
        _combine_body,
        grid=(_N // _RBLK,),
        in_specs=[row(96), row(128), row(32)],
        out_specs=[row(32), row(32), row(32)],
        out_shape=[jax.ShapeDtypeStruct((_N, 32), f32)] * 3,
    )(selfo, accpack, rden)


def _sc_body(mbd_hbm, mfw_hbm, mbw_hbm, src_hbm, dst_hbm, z2_hbm, z1_hbm,
             apack_hbm, deg0_hbm, deg1_hbm,
             acc_s, deg_s, sidx_v, didx_v, rows_v, ones_v, gsem, ssem):
    cid = lax.axis_index("c")
    sid = lax.axis_index("s")

    one16 = jnp.ones((16,), jnp.float32)

    def _init_ones(i, c):
        ones_v[pl.ds(i * 16, 16)] = one16
        return c

    lax.fori_loop(0, 8, _init_ones, 0)

    def _zero_acc():
        base = sid * _TSPAN
        pltpu.sync_copy(z2_hbm.at[pl.ds(base, _TSPAN)],
                        acc_s.at[pl.ds(base, _TSPAN)])

    def _zero_deg():
        base = sid * _TSPAN
        pltpu.sync_copy(z1_hbm.at[pl.ds(base, _TSPAN)],
                        deg_s.at[pl.ds(base, _TSPAN)])

    def _edge_pass(msg_hbm, pass_base, trows, deg_half):
        tile_base = pass_base + sid * trows
        nsup = trows // _SB

        def fire_g(c):
            b = lax.rem(c, 2)
            pltpu.async_copy(msg_hbm.at[sidx_v.at[c]], rows_v.at[b], gsem)

        def drain_g(c):
            b = lax.rem(c, 2)
            pltpu.make_async_copy(msg_hbm.at[sidx_v.at[c]],
                                  rows_v.at[b], gsem).wait()

        def fire_s(c, deg_on):
            b = lax.rem(c, 2)
            pltpu.async_copy(rows_v.at[b], acc_s.at[didx_v.at[c]],
                             ssem, add=True)

            @pl.when(deg_on)
            def _():
                pltpu.async_copy(ones_v, deg_s.at[didx_v.at[c, pl.ds(0, 128)]],
                                 ssem, add=True)
                pltpu.async_copy(ones_v,
                                 deg_s.at[didx_v.at[c, pl.ds(128, 128)]],
                                 ssem, add=True)

        def drain_s(c, deg_on):
            b = lax.rem(c, 2)
            pltpu.make_async_copy(rows_v.at[b], acc_s.at[didx_v.at[c]],
                                  ssem).wait()

            @pl.when(deg_on)
            def _():
                pltpu.make_async_copy(
                    ones_v, deg_s.at[didx_v.at[c, pl.ds(0, 128)]],
                    ssem).wait()
                pltpu.make_async_copy(
                    ones_v, deg_s.at[didx_v.at[c, pl.ds(128, 128)]],
                    ssem).wait()

        def superchunk(sc, carry):
            r0 = tile_base + sc * _SB
            if deg_half == 0:
                deg_on = jnp.bool_(False)
            elif deg_half == 1:
                deg_on = sc < nsup // 2
            else:
                deg_on = sc >= nsup // 2
            pltpu.sync_copy(src_hbm.at[pl.ds(r0, _SB)], sidx_v)
            pltpu.sync_copy(dst_hbm.at[pl.ds(r0, _SB)], didx_v)

            def inner(c, cc):
                @pl.when(c >= 2)
                def _():
                    drain_s(c - 2, deg_on)

                fire_g(c)

                @pl.when(c >= 1)
                def _():
                    drain_g(c - 1)
                    fire_s(c - 1, deg_on)

                return cc

            lax.fori_loop(0, _NCHUNK, inner, 0)
            drain_g(_NCHUNK - 1)
            fire_s(_NCHUNK - 1, deg_on)
            drain_s(_NCHUNK - 2, deg_on)
            drain_s(_NCHUNK - 1, deg_on)
            return carry

        lax.fori_loop(0, nsup, superchunk, 0)

    def _flush(grp):
        base = sid * _TSPAN
        pltpu.sync_copy(acc_s.at[pl.ds(base, _TSPAN)],
                        apack_hbm.at[pl.ds(base, _TSPAN), pl.ds(32 * grp, 32)])

    def _flush_deg(deg_hbm):
        base = sid * _TSPAN
        pltpu.sync_copy(deg_s.at[pl.ds(base, _TSPAN)],
                        deg_hbm.at[pl.ds(base, _TSPAN)])

    @pl.when(cid == 0)
    def _core0():
        _zero_acc()
        _zero_deg()
        plsc.subcore_barrier()
        _edge_pass(mbd_hbm, 0, _TROWS_A, 1)
        plsc.subcore_barrier()
        _flush(0)
        _flush_deg(deg0_hbm)
        plsc.subcore_barrier()
        _zero_acc()
        plsc.subcore_barrier()
        _edge_pass(mbw_hbm, 0, _TROWS_B0, 0)
        plsc.subcore_barrier()
        _flush(2)

    @pl.when(cid == 1)
    def _core1():
        _zero_acc()
        _zero_deg()
        plsc.subcore_barrier()
        _edge_pass(mfw_hbm, 0, _TROWS_A, 2)
        plsc.subcore_barrier()
        _flush(1)
        _flush_deg(deg1_hbm)
        plsc.subcore_barrier()
        _zero_acc()
        plsc.subcore_barrier()
        _edge_pass(mbw_hbm, _BW_SPLIT, _TROWS_B1, 0)
        plsc.subcore_barrier()
        _flush(3)


@functools.lru_cache(maxsize=1)
def _sc_scatter_call():
    mesh = plsc.VectorSubcoreMesh(core_axis_name="c", subcore_axis_name="s",
                                  num_cores=2, num_subcores=_NTILES)
    return pl.kernel(
        _sc_body,
        out_type=[jax.ShapeDtypeStruct((_ACC_ROWS, 128), jnp.float32),
                  jax.ShapeDtypeStruct((_ACC_ROWS,), jnp.float32),
                  jax.ShapeDtypeStruct((_ACC_ROWS,), jnp.float32)],
        mesh=mesh,
        scratch_types=[
            pltpu.VMEM_SHARED((_ACC_ROWS, _MW), jnp.float32),
            pltpu.VMEM_SHARED((_ACC_ROWS,), jnp.float32),
            pltpu.VMEM((_SB, _IW), jnp.int32),
            pltpu.VMEM((_SB, _IW), jnp.int32),
            pltpu.VMEM((2, _IW, _MW), jnp.float32),
            pltpu.VMEM((128,), jnp.float32),
            pltpu.SemaphoreType.DMA,
            pltpu.SemaphoreType.DMA,
        ],
        compiler_params=pltpu.CompilerParams(use_tc_tiling_on_sc=False,
                                            needs_layout_passes=False),
    )


def _assemble_weights(params):
    f32 = jnp.float32
    (wf1, bf1), (wf2, bf2), (wf3, bf3) = params["fw"]
    (wb1, bb1), (wb2, bb2), (wb3, bb3) = params["bw"]
    (ws1, bs1), (ws2, bs2), (ws3, bs3) = params["self"]

    W1 = jnp.zeros((128, 192), f32)
    W1 = W1.at[0:64, 0:64].set(wf1[0:64])
    W1 = W1.at[96:128, 0:64].set(wf1[64:96])
    W1 = W1.at[64:96, 64:128].set(wb1[0:32])
    W1 = W1.at[96:128, 64:128].set(wb1[32:64])
    W1 = W1.at[:, 128:192].set(ws1)
    B1 = jnp.concatenate([bf1, bb1, bs1]).reshape(1, 192)

    W2 = jnp.zeros((192, 192), f32)
    W2 = W2.at[0:64, 0:64].set(wf2)
    W2 = W2.at[64:128, 64:128].set(wb2)
    W2 = W2.at[128:192, 128:192].set(ws2)
    B2 = jnp.concatenate([bf2, bb2, bs2]).reshape(1, 192)

    W3 = jnp.zeros((192, 256), f32)
    W3 = W3.at[0:64, 0:1].set(wf3[:, 0:1])
    W3 = W3.at[0:64, 32:64].set(wf3[:, 1:33])
    W3 = W3.at[0:64, 64:96].set(wf3[:, 33:65])
    W3 = W3.at[64:128, 1:2].set(wb3[:, 0:1])
    W3 = W3.at[64:128, 96:128].set(wb3[:, 1:33])
    W3 = W3.at[64:128, 128:160].set(wb3[:, 33:65])
    W3 = W3.at[128:192, 160:256].set(ws3)
    B3 = jnp.zeros((256,), f32)
    B3 = B3.at[0:1].set(bf3[0:1])
    B3 = B3.at[32:64].set(bf3[1:33])
    B3 = B3.at[64:96].set(bf3[33:65])
    B3 = B3.at[1:2].set(bb3[0:1])
    B3 = B3.at[96:128].set(bb3[1:33])
    B3 = B3.at[128:160].set(bb3[33:65])
    B3 = B3.at[160:256].set(bs3)
    B3 = B3.reshape(1, 256)
    return W1, B1, W2, B2, W3, B3


def kernel(bd_nf, bw_nf, fw_nf, edge_index, params):
    W1, B1, W2, B2, W3, B3 = _assemble_weights(params)

    src = edge_index[0].astype(jnp.int32)
    dst = edge_index[1].astype(jnp.int32)
    pad = _EPAD - _E
    pad_i = jnp.arange(pad, dtype=jnp.int32)
    src3d = jnp.concatenate(
        [src, pad_i % _N]).reshape(_ROWS, _IW)
    dst3d = jnp.concatenate(
        [dst, _TRASH + pad_i % (_ACC_ROWS - _N)]).reshape(_ROWS, _IW)

    mbd, mfw, mbw, selfo = _node_mlps(fw_nf, bw_nf, bd_nf,
                                      W1, B1, W2, B2, W3, B3)
    zeros_acc = jnp.zeros((_ACC_ROWS, _MW), jnp.float32)
    zeros_deg = jnp.zeros((_ACC_ROWS,), jnp.float32)
    accpack, deg0, deg1 = _sc_scatter_call()(mbd, mfw, mbw, src3d, dst3d,
                                             zeros_acc, zeros_deg)
    rden = jnp.broadcast_to(
        (1.0 / jnp.maximum(deg0 + deg1, 1.0))[:, None], (_ACC_ROWS, 32))
    out_bd, out_bw, out_fw = _combine(selfo, accpack, rden)
    return (out_bd, out_bw, out_fw)


# speedup vs baseline: 1.6782x; 1.0246x over previous
import functools

import jax
import jax.numpy as jnp
from jax import lax
from jax.experimental import pallas as pl
from jax.experimental.pallas import tpu as pltpu
from jax.experimental.pallas import tpu_sc as plsc

_N = 50000
_E = 800000

_IW = 256
_MW = 32
_ROWS = 3200
_EPAD = _ROWS * _IW
_NTILES = 16
_SB = 10
_NCHUNK = _SB
_TROWS_A = _ROWS // _NTILES
_BW_SPLIT = 1600
_TROWS_B0 = _BW_SPLIT // _NTILES
_TROWS_B1 = (_ROWS - _BW_SPLIT) // _NTILES
_ACC_ROWS = 50176
_TSPAN = _ACC_ROWS // _NTILES
_TRASH = _N

_RBLK = 5000


def _mlp_body(fw_ref, bw_ref, bd_ref, w1_ref, b1_ref, w2_ref, b2_ref,
              w3_ref, b3_ref, mbd_ref, mfw_ref, mbw_ref, selfo_ref):
    x = jnp.concatenate([fw_ref[...], bw_ref[...], bd_ref[...]], axis=1)
    h = jnp.dot(x, w1_ref[...], preferred_element_type=jnp.float32) + b1_ref[...]
    h = jnp.where(h > 0, h, 0.2 * h)
    h = jnp.dot(h, w2_ref[...], preferred_element_type=jnp.float32) + b2_ref[...]
    h = jnp.where(h > 0, h, 0.2 * h)
    h = jnp.dot(h, w3_ref[...], preferred_element_type=jnp.float32) + b3_ref[...]
    kf = jax.nn.sigmoid(h[:, 0:1])
    kb = jax.nn.sigmoid(h[:, 1:2])
    mbd_ref[...] = h[:, 32:64] * kf + h[:, 96:128] * kb
    mfw_ref[...] = h[:, 64:96] * kf
    mbw_ref[...] = h[:, 128:160] * kb
    selfo_ref[...] = h[:, 160:256]


def _node_mlps(fw_nf, bw_nf, bd_nf, W1, B1, W2, B2, W3, B3):
    f32 = jnp.float32

    def row(w):
        return pl.BlockSpec((_RBLK, w), lambda i: (i, 0))

    def full(a, b):
        return pl.BlockSpec((a, b), lambda i: (0, 0))

    return pl.pallas_call(
        _mlp_body,
        grid=(_N // _RBLK,),
        in_specs=[row(64), row(32), row(32),
                  full(128, 192), full(1, 192),
                  full(192, 192), full(1, 192),
                  full(192, 256), full(1, 256)],
        out_specs=[row(_MW), row(_MW), row(_MW), row(96)],
        out_shape=[jax.ShapeDtypeStruct((_N, _MW), f32),
                   jax.ShapeDtypeStruct((_N, _MW), f32),
                   jax.ShapeDtypeStruct((_N, _MW), f32),
                   jax.ShapeDtypeStruct((_N, 96), f32)],
    )(fw_nf, bw_nf, bd_nf, W1, B1, W2, B2, W3, B3)


def _combine_body(selfo_ref, acc_ref, rden_ref, obd_ref, obw_ref, ofw_ref):
    s = selfo_ref[...]
    a = acc_ref[...]
    r = rden_ref[...]
    obd_ref[...] = jax.nn.sigmoid(s[:, 0:32] + a[:, 0:32] * r)
    obw_ref[...] = jax.nn.sigmoid(
        s[:, 32:64] + (a[:, 64:96] + a[:, 96:128]) * r)
    ofw_ref[...] = jax.nn.sigmoid(s[:, 64:96] + a[:, 32:64] * r)


def _combine(selfo, accpack, rden):
    f32 = jnp.float32

    def row(w):
        return pl.BlockSpec((_RBLK, w), lambda i: (i, 0))

    return pl.pallas_call(
        _combine_body,
        grid=(_N // _RBLK,),
        in_specs=[row(96), row(128), row(32)],
        out_specs=[row(32), row(32), row(32)],
        out_shape=[jax.ShapeDtypeStruct((_N, 32), f32)] * 3,
    )(selfo, accpack, rden)


def _sc_body(mbd_hbm, mfw_hbm, mbw_hbm, src_hbm, dst_hbm, z2_hbm, z1_hbm,
             apack_hbm, deg0_hbm, deg1_hbm,
             acc_s, deg_s, sidx_v, didx_v, rows_v, ones_v, gsem, ssem):
    cid = lax.axis_index("c")
    sid = lax.axis_index("s")

    one16 = jnp.ones((16,), jnp.float32)

    def _init_ones(i, c):
        ones_v[pl.ds(i * 16, 16)] = one16
        return c

    lax.fori_loop(0, 8, _init_ones, 0)

    def _zero_acc():
        base = sid * _TSPAN
        pltpu.sync_copy(z2_hbm.at[pl.ds(base, _TSPAN)],
                        acc_s.at[pl.ds(base, _TSPAN)])

    def _zero_deg():
        base = sid * _TSPAN
        pltpu.sync_copy(z1_hbm.at[pl.ds(base, _TSPAN)],
                        deg_s.at[pl.ds(base, _TSPAN)])

    def _edge_pass(msg_hbm, pass_base, trows, deg_half):
        tile_base = pass_base + sid * trows
        nsup = trows // _SB

        def fire_g(c):
            b = lax.rem(c, 2)
            pltpu.async_copy(msg_hbm.at[sidx_v.at[c]], rows_v.at[b], gsem)

        def drain_g(c):
            b = lax.rem(c, 2)
            pltpu.make_async_copy(msg_hbm.at[sidx_v.at[c]],
                                  rows_v.at[b], gsem).wait()

        def fire_s(c, deg_on):
            b = lax.rem(c, 2)
            pltpu.async_copy(rows_v.at[b], acc_s.at[didx_v.at[c]],
                             ssem, add=True)

            @pl.when(deg_on)
            def _():
                pltpu.async_copy(ones_v, deg_s.at[didx_v.at[c, pl.ds(0, 128)]],
                                 ssem, add=True)
                pltpu.async_copy(ones_v,
                                 deg_s.at[didx_v.at[c, pl.ds(128, 128)]],
                                 ssem, add=True)

        def drain_s(c, deg_on):
            b = lax.rem(c, 2)
            pltpu.make_async_copy(rows_v.at[b], acc_s.at[didx_v.at[c]],
                                  ssem).wait()

            @pl.when(deg_on)
            def _():
                pltpu.make_async_copy(
                    ones_v, deg_s.at[didx_v.at[c, pl.ds(0, 128)]],
                    ssem).wait()
                pltpu.make_async_copy(
                    ones_v, deg_s.at[didx_v.at[c, pl.ds(128, 128)]],
                    ssem).wait()

        def superchunk(sc, carry):
            r0 = tile_base + sc * _SB
            if deg_half == 0:
                deg_on = jnp.bool_(False)
            elif deg_half == 1:
                deg_on = sc < nsup // 2
            else:
                deg_on = sc >= nsup // 2
            pltpu.sync_copy(src_hbm.at[pl.ds(r0, _SB)], sidx_v)
            pltpu.sync_copy(dst_hbm.at[pl.ds(r0, _SB)], didx_v)

            def inner(c, cc):
                @pl.when(c >= 2)
                def _():
                    drain_s(c - 2, deg_on)

                fire_g(c)

                @pl.when(c >= 1)
                def _():
                    drain_g(c - 1)
                    fire_s(c - 1, deg_on)

                return cc

            lax.fori_loop(0, _NCHUNK, inner, 0)
            drain_g(_NCHUNK - 1)
            fire_s(_NCHUNK - 1, deg_on)
            drain_s(_NCHUNK - 2, deg_on)
            drain_s(_NCHUNK - 1, deg_on)
            return carry

        lax.fori_loop(0, nsup, superchunk, 0)

    def _flush(grp):
        base = sid * _TSPAN
        pltpu.sync_copy(acc_s.at[pl.ds(base, _TSPAN)],
                        apack_hbm.at[pl.ds(base, _TSPAN), pl.ds(32 * grp, 32)])

    def _flush_deg(deg_hbm):
        base = sid * _TSPAN
        pltpu.sync_copy(deg_s.at[pl.ds(base, _TSPAN)],
                        deg_hbm.at[pl.ds(base, _TSPAN)])

    @pl.when(cid == 0)
    def _core0():
        _zero_acc()
        _zero_deg()
        plsc.subcore_barrier()
        _edge_pass(mbd_hbm, 0, _TROWS_A, 1)
        plsc.subcore_barrier()
        _flush(0)
        _flush_deg(deg0_hbm)
        plsc.subcore_barrier()
        _zero_acc()
        plsc.subcore_barrier()
        _edge_pass(mbw_hbm, 0, _TROWS_B0, 0)
        plsc.subcore_barrier()
        _flush(2)

    @pl.when(cid == 1)
    def _core1():
        _zero_acc()
        _zero_deg()
        plsc.subcore_barrier()
        _edge_pass(mfw_hbm, 0, _TROWS_A, 2)
        plsc.subcore_barrier()
        _flush(1)
        _flush_deg(deg1_hbm)
        plsc.subcore_barrier()
        _zero_acc()
        plsc.subcore_barrier()
        _edge_pass(mbw_hbm, _BW_SPLIT, _TROWS_B1, 0)
        plsc.subcore_barrier()
        _flush(3)


@functools.lru_cache(maxsize=1)
def _sc_scatter_call():
    mesh = plsc.VectorSubcoreMesh(core_axis_name="c", subcore_axis_name="s",
                                  num_cores=2, num_subcores=_NTILES)
    return pl.kernel(
        _sc_body,
        out_type=[jax.ShapeDtypeStruct((_ACC_ROWS, 128), jnp.float32),
                  jax.ShapeDtypeStruct((_ACC_ROWS,), jnp.float32),
                  jax.ShapeDtypeStruct((_ACC_ROWS,), jnp.float32)],
        mesh=mesh,
        scratch_types=[
            pltpu.VMEM_SHARED((_ACC_ROWS, _MW), jnp.float32),
            pltpu.VMEM_SHARED((_ACC_ROWS,), jnp.float32),
            pltpu.VMEM((_SB, _IW), jnp.int32),
            pltpu.VMEM((_SB, _IW), jnp.int32),
            pltpu.VMEM((2, _IW, _MW), jnp.float32),
            pltpu.VMEM((128,), jnp.float32),
            pltpu.SemaphoreType.DMA,
            pltpu.SemaphoreType.DMA,
        ],
        compiler_params=pltpu.CompilerParams(use_tc_tiling_on_sc=False,
                                            needs_layout_passes=False),
    )


def _assemble_weights(params):
    f32 = jnp.float32
    (wf1, bf1), (wf2, bf2), (wf3, bf3) = params["fw"]
    (wb1, bb1), (wb2, bb2), (wb3, bb3) = params["bw"]
    (ws1, bs1), (ws2, bs2), (ws3, bs3) = params["self"]

    W1 = jnp.zeros((128, 192), f32)
    W1 = W1.at[0:64, 0:64].set(wf1[0:64])
    W1 = W1.at[96:128, 0:64].set(wf1[64:96])
    W1 = W1.at[64:96, 64:128].set(wb1[0:32])
    W1 = W1.at[96:128, 64:128].set(wb1[32:64])
    W1 = W1.at[:, 128:192].set(ws1)
    B1 = jnp.concatenate([bf1, bb1, bs1]).reshape(1, 192)

    W2 = jnp.zeros((192, 192), f32)
    W2 = W2.at[0:64, 0:64].set(wf2)
    W2 = W2.at[64:128, 64:128].set(wb2)
    W2 = W2.at[128:192, 128:192].set(ws2)
    B2 = jnp.concatenate([bf2, bb2, bs2]).reshape(1, 192)

    W3 = jnp.zeros((192, 256), f32)
    W3 = W3.at[0:64, 0:1].set(wf3[:, 0:1])
    W3 = W3.at[0:64, 32:64].set(wf3[:, 1:33])
    W3 = W3.at[0:64, 64:96].set(wf3[:, 33:65])
    W3 = W3.at[64:128, 1:2].set(wb3[:, 0:1])
    W3 = W3.at[64:128, 96:128].set(wb3[:, 1:33])
    W3 = W3.at[64:128, 128:160].set(wb3[:, 33:65])
    W3 = W3.at[128:192, 160:256].set(ws3)
    B3 = jnp.zeros((256,), f32)
    B3 = B3.at[0:1].set(bf3[0:1])
    B3 = B3.at[32:64].set(bf3[1:33])
    B3 = B3.at[64:96].set(bf3[33:65])
    B3 = B3.at[1:2].set(bb3[0:1])
    B3 = B3.at[96:128].set(bb3[1:33])
    B3 = B3.at[128:160].set(bb3[33:65])
    B3 = B3.at[160:256].set(bs3)
    B3 = B3.reshape(1, 256)
    return W1, B1, W2, B2, W3, B3


def kernel(bd_nf, bw_nf, fw_nf, edge_index, params):
    W1, B1, W2, B2, W3, B3 = _assemble_weights(params)

    src = edge_index[0].astype(jnp.int32)
    dst = edge_index[1].astype(jnp.int32)
    pad = _EPAD - _E
    pad_i = jnp.arange(pad, dtype=jnp.int32)
    src3d = jnp.concatenate(
        [src, pad_i % _N]).reshape(_ROWS, _IW)
    dst3d = jnp.concatenate(
        [dst, _TRASH + pad_i % (_ACC_ROWS - _N)]).reshape(_ROWS, _IW)

    mbd, mfw, mbw, selfo = _node_mlps(fw_nf, bw_nf, bd_nf,
                                      W1, B1, W2, B2, W3, B3)
    zeros_acc = jnp.zeros((_ACC_ROWS, _MW), jnp.float32)
    zeros_deg = jnp.zeros((_ACC_ROWS,), jnp.float32)
    accpack, deg0, deg1 = _sc_scatter_call()(mbd, mfw, mbw, src3d, dst3d,
                                             zeros_acc, zeros_deg)
    rden = jnp.broadcast_to(
        (1.0 / jnp.maximum(deg0 + deg1, 1.0))[:, None], (_ACC_ROWS, 32))
    out_bd, out_bw, out_fw = _combine(selfo, accpack, rden)
    return (out_bd, out_bw, out_fw)
